# Initial kernel scaffold; baseline (speedup 1.0000x reference)
import functools

import jax
import jax.numpy as jnp
from jax import lax
from jax.experimental import pallas as pl
from jax.experimental.pallas import tpu as pltpu
from jax.experimental.pallas import tpu_sc as plsc

N_PERF, N_ART, N_SONG = 50000, 10000, 10000
E = 200000
D_EMB, D_HID, D_PROJ = 128, 256, 128

NC, NS = 2, 16
NW = NC * NS
NPA_PERF = 51200
NPA_SM = 10240
E_PAD = 200704
ES = E_PAD // NS
NV = ES // 16
EW = E_PAD // NW
NVW = EW // 16

MESH = plsc.VectorSubcoreMesh(core_axis_name="c", subcore_axis_name="s")


@functools.lru_cache(None)
def _gather_rows(V, B, D):
    b_per_w = B // NW
    nch = b_per_w // 64

    def body(tab_hbm, ids_hbm, out_hbm, idx_v, rows_v, sem):
        c = lax.axis_index("c")
        s = lax.axis_index("s")
        w = s * NC + c
        base = w * b_per_w
        pltpu.sync_copy(ids_hbm.at[pl.ds(base, b_per_w)], idx_v)
        for j in range(nch):
            cp = pltpu.async_copy(
                tab_hbm.at[idx_v.at[pl.ds(j * 64, 64)]], rows_v, sem)
            cp.wait()
            pltpu.sync_copy(rows_v, out_hbm.at[pl.ds(base + j * 64, 64), :])

    return pl.kernel(
        body,
        out_type=jax.ShapeDtypeStruct((B, D), jnp.float32),
        mesh=MESH,
        scratch_types=[
            pltpu.VMEM((b_per_w,), jnp.int32),
            pltpu.VMEM((64, D), jnp.float32),
            pltpu.SemaphoreType.DMA,
        ],
    )


def _hist_kernel():
    NPAS = (NPA_PERF, NPA_SM, NPA_SM, NPA_PERF)

    def body(d0, d1, d2, d3, o0, o1, o2, o3, dst_v, cnt_v, zb_v, cnt_sp):
        c = lax.axis_index("c")
        s = lax.axis_index("s")
        w = s * NC + c
        ones = jnp.ones((16,), jnp.float32)
        zvec = jnp.zeros((16,), jnp.float32)

        def zloop(i, _):
            zb_v[pl.ds(i * 16, 16)] = zvec
            return 0

        lax.fori_loop(0, zb_v.shape[0] // 16, zloop, 0)

        for dst_hbm, out_hbm, npa in zip((d0, d1, d2, d3),
                                         (o0, o1, o2, o3), NPAS):
            share = npa // NS

            def czero(i, _):
                cnt_v[pl.ds(i * 16, 16)] = zvec
                return 0

            lax.fori_loop(0, npa // 16, czero, 0)
            pltpu.sync_copy(zb_v.at[pl.ds(0, share)],
                            cnt_sp.at[pl.ds(s * share, share)])
            pltpu.sync_copy(dst_hbm.at[pl.ds(w * EW, EW)], dst_v)

            def hloop(i, _):
                d = dst_v[pl.ds(i * 16, 16)]
                plsc.addupdate_scatter(cnt_v, [d], ones)
                return 0

            lax.fori_loop(0, NVW, hloop, 0)
            plsc.subcore_barrier()
            pltpu.sync_copy(cnt_v.at[pl.ds(0, npa)], cnt_sp.at[pl.ds(0, npa)],
                            add=True)
            plsc.subcore_barrier()
            pltpu.sync_copy(cnt_sp.at[pl.ds(s * share, share)],
                            out_hbm.at[c, pl.ds(s * share, share)])

    return pl.kernel(
        body,
        out_type=tuple(jax.ShapeDtypeStruct((NC, n), jnp.float32)
                       for n in NPAS),
        mesh=MESH,
        scratch_types=[
            pltpu.VMEM((EW,), jnp.int32),
            pltpu.VMEM((NPA_PERF,), jnp.float32),
            pltpu.VMEM((NPA_PERF // NS,), jnp.float32),
            pltpu.VMEM_SHARED((NPA_PERF,), jnp.float32),
        ],
    )


@functools.lru_cache(None)
def _segsum(V_src, NPA_dst, D, CH, npasses):
    rps = CH // NS
    NB_MAX = (ES + 128) // 128

    def body(x_hbm, src_hbm, dst_hbm, agg_hbm,
             src_v, dst_v, cs2, cd2, msg0, msg1, wb_v, acc_sp,
             sem0, sem1):
        c = lax.axis_index("c")
        s = lax.axis_index("s")
        msgs = (msg0, msg1)
        sems = (sem0, sem1)
        pltpu.sync_copy(src_hbm.at[pl.ds(s * ES, ES)], src_v)
        pltpu.sync_copy(dst_hbm.at[pl.ds(s * ES, ES)], dst_v)
        zvec = jnp.zeros((16,), jnp.float32)
        for r in range(16):
            for q in range(D // 16):
                wb_v[r, pl.ds(q * 16, 16)] = zvec
        iota = lax.iota(jnp.int32, 16)
        zeros_i = jnp.zeros((16,), jnp.int32)
        dump_i = jnp.full((16,), CH, jnp.int32)

        for p in range(npasses):
            lo = (p * NC + c) * CH
            for j in range(rps // 16):
                pltpu.sync_copy(wb_v.at[pl.ds(0, 16), :],
                                acc_sp.at[pl.ds(s * rps + j * 16, 16), :])
            plsc.subcore_barrier()

            def fbody(i, cnt):
                d = dst_v[pl.ds(i * 16, 16)]
                sv = src_v[pl.ds(i * 16, 16)]
                m = (d >= lo) & (d < lo + CH)
                pos = cnt + plsc.cumsum(jnp.where(m, 1, 0)) - 1
                hi_ = lax.shift_right_logical(pos, 7)
                lo_ = lax.bitwise_and(pos, 127)
                plsc.store_scatter(cs2, [hi_, lo_], sv, mask=m)
                plsc.store_scatter(cd2, [hi_, lo_], d - lo, mask=m)
                return cnt + jnp.sum(m.astype(jnp.int32))

            cnt = lax.fori_loop(0, NV, fbody, 0)
            for j in range(8):
                pp = cnt + j * 16 + iota
                hi_ = lax.shift_right_logical(pp, 7)
                lo_ = lax.bitwise_and(pp, 127)
                plsc.store_scatter(cs2, [hi_, lo_], zeros_i)
                plsc.store_scatter(cd2, [hi_, lo_], dump_i)
            nb = (cnt + 127) // 128

            @pl.when(nb > 0)
            def _():
                pltpu.async_copy(x_hbm.at[cs2.at[0]], msgs[0], sems[0])

            def bbody(t, _):
                for k in range(2):
                    b = 2 * t + k

                    @pl.when(b + 1 < nb)
                    def _():
                        pltpu.async_copy(x_hbm.at[cs2.at[b + 1]],
                                         msgs[1 - k], sems[1 - k])

                    @pl.when(b < nb)
                    def _():
                        pltpu.make_async_copy(
                            x_hbm.at[cs2.at[b]], msgs[k], sems[k]).wait()
                        pltpu.sync_copy(msgs[k], acc_sp.at[cd2.at[b]],
                                        add=True)
                return 0

            lax.fori_loop(0, (nb + 1) // 2, bbody, 0)
            plsc.subcore_barrier()

            for j in range(rps // 80):
                row = s * rps + j * 80
                pltpu.sync_copy(acc_sp.at[pl.ds(row, 80), :],
                                wb_v.at[pl.ds(0, 80), :])
                pltpu.sync_copy(wb_v.at[pl.ds(0, 80), :],
                                agg_hbm.at[pl.ds(lo + row, 80), :])

    return pl.kernel(
        body,
        out_type=jax.ShapeDtypeStruct((NC * CH * npasses, D), jnp.float32),
        mesh=MESH,
        scratch_types=[
            pltpu.VMEM((ES,), jnp.int32),
            pltpu.VMEM((ES,), jnp.int32),
            pltpu.VMEM((NB_MAX, 128), jnp.int32),
            pltpu.VMEM((NB_MAX, 128), jnp.int32),
            pltpu.VMEM((128, D), jnp.float32),
            pltpu.VMEM((128, D), jnp.float32),
            pltpu.VMEM((80, D), jnp.float32),
            pltpu.VMEM_SHARED((CH + 16, D), jnp.float32),
            pltpu.SemaphoreType.DMA,
            pltpu.SemaphoreType.DMA,
        ],
    )


_PREC = lax.Precision.HIGHEST


@functools.lru_cache(None)
def _combine(n_rows, Din, n_edges, relu, head):
    R = 256
    Dout = D_PROJ if head else D_HID

    def body(*refs):
        i = 0
        x_ref = refs[i]; i += 1
        aggs, recips, wls, wrs, bs = [], [], [], [], []
        for _ in range(n_edges):
            aggs.append(refs[i]); i += 1
            recips.append(refs[i]); i += 1
            wls.append(refs[i]); i += 1
            wrs.append(refs[i]); i += 1
            bs.append(refs[i]); i += 1
        if head:
            g_ref = refs[i]; b_ref = refs[i + 1]
            p1_ref = refs[i + 2]; pb1_ref = refs[i + 3]
            p2_ref = refs[i + 4]; pb2_ref = refs[i + 5]
            i += 6
        out_ref = refs[i]

        x = x_ref[...]
        acc = None
        for e in range(n_edges):
            a = aggs[e][...] * recips[e][...]
            h = (lax.dot_general(a, wls[e][...], (((1,), (1,)), ((), ())),
                                 precision=_PREC,
                                 preferred_element_type=jnp.float32)
                 + lax.dot_general(x, wrs[e][...], (((1,), (1,)), ((), ())),
                                   precision=_PREC,
                                   preferred_element_type=jnp.float32)
                 + bs[e][...])
            nrm = jnp.maximum(
                jnp.sqrt(jnp.sum(h * h, axis=-1, keepdims=True)), 1e-12)
            o = h / nrm
            acc = o if acc is None else acc + o
        if relu:
            acc = jnp.maximum(acc, 0.0)
        if head:
            mu = jnp.mean(acc, axis=-1, keepdims=True)
            var = jnp.mean((acc - mu) ** 2, axis=-1, keepdims=True)
            hn = (acc - mu) / jnp.sqrt(var + 1e-5) * g_ref[...] + b_ref[...]
            z = jnp.maximum(
                lax.dot_general(hn, p1_ref[...], (((1,), (1,)), ((), ())),
                                precision=_PREC,
                                preferred_element_type=jnp.float32)
                + pb1_ref[...], 0.0)
            acc = (lax.dot_general(z, p2_ref[...], (((1,), (1,)), ((), ())),
                                   precision=_PREC,
                                   preferred_element_type=jnp.float32)
                   + pb2_ref[...])
        out_ref[...] = acc

    row_spec = pl.BlockSpec((R, Din), lambda i: (i, 0))
    col1_spec = pl.BlockSpec((R, 1), lambda i: (i, 0))
    full = lambda *shape: pl.BlockSpec(shape, lambda i: (0,) * len(shape))
    in_specs = [row_spec]
    for _ in range(n_edges):
        in_specs += [row_spec, col1_spec,
                     full(D_HID, Din), full(D_HID, Din), full(1, D_HID)]
    if head:
        in_specs += [full(1, D_HID), full(1, D_HID),
                     full(D_PROJ, D_HID), full(1, D_PROJ),
                     full(D_PROJ, D_PROJ), full(1, D_PROJ)]

    return pl.pallas_call(
        body,
        grid=(n_rows // R,),
        in_specs=in_specs,
        out_specs=pl.BlockSpec((R, Dout), lambda i: (i, 0)),
        out_shape=jax.ShapeDtypeStruct((n_rows, Dout), jnp.float32),
    )


def _pad1(a, n, val):
    return jnp.pad(a, (0, n - a.shape[0]), constant_values=val)


def kernel(emb_perf, emb_artist, emb_song, Wl0, bl0, Wr0, br0, Wl1, bl1,
           Wr1, br1, Wl2, bl2, Wr2, br2, ln_g, ln_b, P1, pb1, P2, pb2,
           n_id_perf, n_id_artist, n_id_song, src_ap, dst_ap, src_pa,
           dst_pa, src_ps, dst_ps, src_sp, dst_sp):
    V_PERF, V_ART, V_SONG = (emb_perf.shape[0], emb_artist.shape[0],
                             emb_song.shape[0])
    x_perf = _gather_rows(V_PERF, NPA_PERF, D_EMB)(
        emb_perf, _pad1(n_id_perf, NPA_PERF, 0))
    x_art = _gather_rows(V_ART, NPA_SM, D_EMB)(
        emb_artist, _pad1(n_id_artist, NPA_SM, 0))
    x_song = _gather_rows(V_SONG, NPA_SM, D_EMB)(
        emb_song, _pad1(n_id_song, NPA_SM, 0))

    sap, dap = _pad1(src_ap, E_PAD, 0), _pad1(dst_ap, E_PAD, N_PERF)
    spa, dpa = _pad1(src_pa, E_PAD, 0), _pad1(dst_pa, E_PAD, N_ART)
    sps, dps = _pad1(src_ps, E_PAD, 0), _pad1(dst_ps, E_PAD, N_SONG)
    ssp, dsp = _pad1(src_sp, E_PAD, 0), _pad1(dst_sp, E_PAD, N_PERF)

    c_ap, c_pa, c_ps, c_sp = _hist_kernel()(dap, dpa, dps, dsp)
    rec = lambda c: (1.0 / jnp.maximum(c[0] + c[1], 1.0)).reshape(-1, 1)
    r_ap, r_pa, r_ps, r_sp = rec(c_ap), rec(c_pa), rec(c_ps), rec(c_sp)

    Wls = (Wl0, Wl1, Wl2)
    bls = (bl0, bl1, bl2)
    Wrs = (Wr0, Wr1, Wr2)
    brs = (br0, br1, br2)

    for li in range(3):
        Din = D_EMB if li == 0 else D_HID
        ch_perf, np_perf = (12800, 2) if li == 0 else (6400, 4)
        seg_to_perf = _segsum(NPA_SM, NPA_PERF, Din, ch_perf, np_perf)
        seg_to_sm = _segsum(NPA_PERF, NPA_SM, Din, 5120, 1)
        agg_ap = seg_to_perf(x_art, sap, dap)
        agg_sp = seg_to_perf(x_song, ssp, dsp)
        agg_pa = seg_to_sm(x_perf, spa, dpa)
        agg_ps = seg_to_sm(x_perf, sps, dps)

        Wl, bl, Wr, br = Wls[li], bls[li], Wrs[li], brs[li]
        head = li == 2
        relu = not head
        bsum = lambda e: (bl[e] + br[e]).reshape(1, D_HID)
        head_args = lambda ti: ((ln_g[ti].reshape(1, D_HID),
                                 ln_b[ti].reshape(1, D_HID),
                                 P1[ti], pb1[ti].reshape(1, D_PROJ),
                                 P2[ti], pb2[ti].reshape(1, D_PROJ))
                                if head else ())
        x_perf = _combine(NPA_PERF, Din, 2, relu, head)(
            x_perf, agg_ap[:NPA_PERF], r_ap, Wl[0], Wr[0], bsum(0),
            agg_sp[:NPA_PERF], r_sp, Wl[3], Wr[3], bsum(3), *head_args(0))
        x_art = _combine(NPA_SM, Din, 1, relu, head)(
            x_art, agg_pa[:NPA_SM], r_pa, Wl[1], Wr[1], bsum(1),
            *head_args(1))
        x_song = _combine(NPA_SM, Din, 1, relu, head)(
            x_song, agg_ps[:NPA_SM], r_ps, Wl[2], Wr[2], bsum(2),
            *head_args(2))

    return (x_perf[:N_PERF], x_art[:N_ART], x_song[:N_SONG])


# speedup vs baseline: 1.0795x; 1.0795x over previous
import functools

import jax
import jax.numpy as jnp
from jax import lax
from jax.experimental import pallas as pl
from jax.experimental.pallas import tpu as pltpu
from jax.experimental.pallas import tpu_sc as plsc

N_PERF, N_ART, N_SONG = 50000, 10000, 10000
E = 200000
D_EMB, D_HID, D_PROJ = 128, 256, 128

NC, NS = 2, 16
NW = NC * NS
NPA_PERF = 51200
NPA_SM = 10240
E_PAD = 200704
ES = E_PAD // NS
NV = ES // 16
EW = E_PAD // NW
NVW = EW // 16

MESH = plsc.VectorSubcoreMesh(core_axis_name="c", subcore_axis_name="s")


@functools.lru_cache(None)
def _gather_rows(V, B, D):
    b_per_w = B // NW
    nch = b_per_w // 64

    def body(tab_hbm, ids_hbm, out_hbm, idx_v, rows_v, sem):
        c = lax.axis_index("c")
        s = lax.axis_index("s")
        w = s * NC + c
        base = w * b_per_w
        pltpu.sync_copy(ids_hbm.at[pl.ds(base, b_per_w)], idx_v)
        for j in range(nch):
            cp = pltpu.async_copy(
                tab_hbm.at[idx_v.at[pl.ds(j * 64, 64)]], rows_v, sem)
            cp.wait()
            pltpu.sync_copy(rows_v, out_hbm.at[pl.ds(base + j * 64, 64), :])

    return pl.kernel(
        body,
        out_type=jax.ShapeDtypeStruct((B, D), jnp.float32),
        mesh=MESH,
        scratch_types=[
            pltpu.VMEM((b_per_w,), jnp.int32),
            pltpu.VMEM((64, D), jnp.float32),
            pltpu.SemaphoreType.DMA,
        ],
        compiler_params=pltpu.CompilerParams(needs_layout_passes=False),
    )


def _hist_kernel():
    NPAS = (NPA_PERF, NPA_SM, NPA_SM, NPA_PERF)

    def body(d0, d1, d2, d3, o0, o1, o2, o3, dst_v, cnt_v, acc_v, tmp_v,
             cnt_sp):
        c = lax.axis_index("c")
        s = lax.axis_index("s")
        w = s * NC + c
        ones = jnp.ones((16,), jnp.float32)
        zvec = jnp.zeros((16,), jnp.float32)

        for dst_hbm, out_hbm, npa in zip((d0, d1, d2, d3),
                                         (o0, o1, o2, o3), NPAS):
            share = npa // NS

            def czero(i, _):
                cnt_v[pl.ds(i * 16, 16)] = zvec
                return 0

            lax.fori_loop(0, npa // 16, czero, 0)
            pltpu.sync_copy(dst_hbm.at[pl.ds(w * EW, EW)], dst_v)

            def hloop(i, _):
                d = dst_v[pl.ds(i * 16, 16)]
                plsc.addupdate_scatter(cnt_v, [d], ones)
                return 0

            lax.fori_loop(0, NVW, hloop, 0)
            pltpu.sync_copy(cnt_v.at[pl.ds(0, npa)],
                            cnt_sp.at[s, pl.ds(0, npa)])
            plsc.subcore_barrier()
            pltpu.sync_copy(cnt_sp.at[0, pl.ds(s * share, share)],
                            acc_v.at[pl.ds(0, share)])
            for t in range(1, NS):
                pltpu.sync_copy(cnt_sp.at[t, pl.ds(s * share, share)],
                                tmp_v.at[pl.ds(0, share)])

                def aloop(i, _):
                    acc_v[pl.ds(i * 16, 16)] = (acc_v[pl.ds(i * 16, 16)]
                                                + tmp_v[pl.ds(i * 16, 16)])
                    return 0

                lax.fori_loop(0, share // 16, aloop, 0)
            pltpu.sync_copy(acc_v.at[pl.ds(0, share)],
                            out_hbm.at[c, pl.ds(s * share, share)])
            plsc.subcore_barrier()

    return pl.kernel(
        body,
        out_type=tuple(jax.ShapeDtypeStruct((NC, n), jnp.float32)
                       for n in NPAS),
        mesh=MESH,
        scratch_types=[
            pltpu.VMEM((EW,), jnp.int32),
            pltpu.VMEM((NPA_PERF,), jnp.float32),
            pltpu.VMEM((NPA_PERF // NS,), jnp.float32),
            pltpu.VMEM((NPA_PERF // NS,), jnp.float32),
            pltpu.VMEM_SHARED((NS, NPA_PERF), jnp.float32),
        ],
        compiler_params=pltpu.CompilerParams(needs_layout_passes=False),
    )


@functools.lru_cache(None)
def _segsum(D, NPA_dst):
    BATCH = 64
    LOGB = 6
    SEC = 1792
    NSEC = ES // SEC
    NB_MAX = SEC // BATCH + 1
    CH = min(NPA_dst // NC, 12800)
    PASSES = NPA_dst // (CH * NC)
    RPS = CH // NS

    def body(x_hbm, src_hbm, dst_hbm, agg_hbm, src_s, dst_s, cs2, cd2,
             msg0, msg1, acc_sp, sem0, sem1):
        c = lax.axis_index("c")
        s = lax.axis_index("s")
        zvec = jnp.zeros((16,), jnp.float32)
        iota = lax.iota(jnp.int32, 16)
        zeros_i = jnp.zeros((16,), jnp.int32)
        dump_i = jnp.full((16,), CH, jnp.int32)
        msgs = (msg0, msg1)
        sems = (sem0, sem1)

        def fire(b, msg, sem):
            pltpu.async_copy(x_hbm.at[cs2.at[b]], msg, sem)

        def drain(b, msg, sem):
            pltpu.make_async_copy(x_hbm.at[cs2.at[b]], msg, sem).wait()
            pltpu.sync_copy(msg, acc_sp.at[cd2.at[b]], add=True)

        for p in range(PASSES):
            lo = (p * NC + c) * CH

            def zrow(r, _):
                for q in range(D // 16):
                    msg0[r, pl.ds(q * 16, 16)] = zvec
                return 0

            lax.fori_loop(0, 16, zrow, 0)

            def zacc(j, _):
                pltpu.sync_copy(msg0.at[pl.ds(0, 16), :],
                                acc_sp.at[pl.ds(s * RPS + j * 16, 16), :])
                return 0

            lax.fori_loop(0, RPS // 16, zacc, 0)
            plsc.subcore_barrier()

            def sloop(sec, _):
                off = s * ES + sec * SEC
                pltpu.sync_copy(src_hbm.at[pl.ds(off, SEC)], src_s)
                pltpu.sync_copy(dst_hbm.at[pl.ds(off, SEC)], dst_s)

                def fbody(i, cnt):
                    d = dst_s[pl.ds(i * 16, 16)]
                    sv = src_s[pl.ds(i * 16, 16)]
                    m = (d >= lo) & (d < lo + CH)
                    pos = cnt + plsc.cumsum(jnp.where(m, 1, 0)) - 1
                    hi_ = lax.shift_right_logical(pos, LOGB)
                    lo_ = lax.bitwise_and(pos, BATCH - 1)
                    plsc.store_scatter(cs2, [hi_, lo_], sv, mask=m)
                    plsc.store_scatter(cd2, [hi_, lo_], d - lo, mask=m)
                    return cnt + jnp.sum(m.astype(jnp.int32))

                cnt = lax.fori_loop(0, SEC // 16, fbody, 0)
                for j in range(BATCH // 16):
                    pp = cnt + j * 16 + iota
                    hi_ = lax.shift_right_logical(pp, LOGB)
                    lo_ = lax.bitwise_and(pp, BATCH - 1)
                    plsc.store_scatter(cs2, [hi_, lo_], zeros_i)
                    plsc.store_scatter(cd2, [hi_, lo_], dump_i)
                nb = (cnt + BATCH - 1) // BATCH

                @pl.when(nb > 0)
                def _():
                    fire(0, msg0, sem0)

                def bloop(t, _):
                    for k in range(2):
                        b = 2 * t + k

                        @pl.when(b + 1 < nb)
                        def _():
                            fire(b + 1, msgs[1 - k], sems[1 - k])

                        @pl.when(b < nb)
                        def _():
                            drain(b, msgs[k], sems[k])
                    return 0

                lax.fori_loop(0, (nb + 1) // 2, bloop, 0)
                return 0

            lax.fori_loop(0, NSEC, sloop, 0)
            plsc.subcore_barrier()
            pltpu.sync_copy(acc_sp.at[pl.ds(s * RPS, RPS), :],
                            agg_hbm.at[pl.ds(lo + s * RPS, RPS), :])

    return pl.kernel(
        body,
        out_type=jax.ShapeDtypeStruct((NPA_dst, D), jnp.float32),
        mesh=MESH,
        scratch_types=[
            pltpu.VMEM((SEC,), jnp.int32),
            pltpu.VMEM((SEC,), jnp.int32),
            pltpu.VMEM((NB_MAX, BATCH), jnp.int32),
            pltpu.VMEM((NB_MAX, BATCH), jnp.int32),
            pltpu.VMEM((BATCH, D), jnp.float32),
            pltpu.VMEM((BATCH, D), jnp.float32),
            pltpu.VMEM_SHARED((CH + 16, D), jnp.float32),
            pltpu.SemaphoreType.DMA,
            pltpu.SemaphoreType.DMA,
        ],
        compiler_params=pltpu.CompilerParams(needs_layout_passes=False),
    )


_PREC = lax.Precision.HIGHEST


@functools.lru_cache(None)
def _combine(n_rows, P, n_edges, relu, head):
    R = 256
    Din = 128 * P

    def body(*refs):
        i = 0
        xps = refs[:P]; i += P
        aggs, recips, wls, wrs, bs = [], [], [], [], []
        for _ in range(n_edges):
            aggs.append(refs[i:i + P]); i += P
            recips.append(refs[i]); i += 1
            wls.append(refs[i]); i += 1
            wrs.append(refs[i]); i += 1
            bs.append(refs[i]); i += 1
        if head:
            g_ref = refs[i]; b_ref = refs[i + 1]
            p1_ref = refs[i + 2]; pb1_ref = refs[i + 3]
            p2_ref = refs[i + 4]; pb2_ref = refs[i + 5]
            i += 6
        outs = refs[i:]

        cat = lambda parts: (parts[0][...] if P == 1 else
                             jnp.concatenate([p[...] for p in parts], axis=1))
        x = cat(xps)
        acc = None
        for e in range(n_edges):
            a = cat(aggs[e]) * recips[e][...]
            h = (lax.dot_general(a, wls[e][...], (((1,), (1,)), ((), ())),
                                 precision=_PREC,
                                 preferred_element_type=jnp.float32)
                 + lax.dot_general(x, wrs[e][...], (((1,), (1,)), ((), ())),
                                   precision=_PREC,
                                   preferred_element_type=jnp.float32)
                 + bs[e][...])
            nrm = jnp.maximum(
                jnp.sqrt(jnp.sum(h * h, axis=-1, keepdims=True)), 1e-12)
            o = h / nrm
            acc = o if acc is None else acc + o
        if relu:
            acc = jnp.maximum(acc, 0.0)
        if head:
            mu = jnp.mean(acc, axis=-1, keepdims=True)
            var = jnp.mean((acc - mu) ** 2, axis=-1, keepdims=True)
            hn = (acc - mu) / jnp.sqrt(var + 1e-5) * g_ref[...] + b_ref[...]
            z = jnp.maximum(
                lax.dot_general(hn, p1_ref[...], (((1,), (1,)), ((), ())),
                                precision=_PREC,
                                preferred_element_type=jnp.float32)
                + pb1_ref[...], 0.0)
            outs[0][...] = (lax.dot_general(z, p2_ref[...],
                                            (((1,), (1,)), ((), ())),
                                            precision=_PREC,
                                            preferred_element_type=jnp.float32)
                            + pb2_ref[...])
        else:
            outs[0][...] = acc[:, :128]
            outs[1][...] = acc[:, 128:]

    half_spec = pl.BlockSpec((R, 128), lambda i: (i, 0))
    col1_spec = pl.BlockSpec((R, 1), lambda i: (i, 0))
    full = lambda *shape: pl.BlockSpec(shape, lambda i: (0,) * len(shape))
    in_specs = [half_spec] * P
    for _ in range(n_edges):
        in_specs += [half_spec] * P
        in_specs += [col1_spec,
                     full(D_HID, Din), full(D_HID, Din), full(1, D_HID)]
    if head:
        in_specs += [full(1, D_HID), full(1, D_HID),
                     full(D_PROJ, D_HID), full(1, D_PROJ),
                     full(D_PROJ, D_PROJ), full(1, D_PROJ)]
        out_specs = pl.BlockSpec((R, D_PROJ), lambda i: (i, 0))
        out_shape = jax.ShapeDtypeStruct((n_rows, D_PROJ), jnp.float32)
    else:
        out_specs = (half_spec, half_spec)
        out_shape = (jax.ShapeDtypeStruct((n_rows, 128), jnp.float32),
                     jax.ShapeDtypeStruct((n_rows, 128), jnp.float32))

    return pl.pallas_call(
        body,
        grid=(n_rows // R,),
        in_specs=in_specs,
        out_specs=out_specs,
        out_shape=out_shape,
    )


def _pad1(a, n, val):
    return jnp.pad(a, (0, n - a.shape[0]), constant_values=val)


def kernel(emb_perf, emb_artist, emb_song, Wl0, bl0, Wr0, br0, Wl1, bl1,
           Wr1, br1, Wl2, bl2, Wr2, br2, ln_g, ln_b, P1, pb1, P2, pb2,
           n_id_perf, n_id_artist, n_id_song, src_ap, dst_ap, src_pa,
           dst_pa, src_ps, dst_ps, src_sp, dst_sp):
    V_PERF, V_ART, V_SONG = (emb_perf.shape[0], emb_artist.shape[0],
                             emb_song.shape[0])
    xp = (_gather_rows(V_PERF, NPA_PERF, D_EMB)(
        emb_perf, _pad1(n_id_perf, NPA_PERF, 0)),)
    xa = (_gather_rows(V_ART, NPA_SM, D_EMB)(
        emb_artist, _pad1(n_id_artist, NPA_SM, 0)),)
    xs = (_gather_rows(V_SONG, NPA_SM, D_EMB)(
        emb_song, _pad1(n_id_song, NPA_SM, 0)),)

    sap, dap = _pad1(src_ap, E_PAD, 0), _pad1(dst_ap, E_PAD, N_PERF)
    spa, dpa = _pad1(src_pa, E_PAD, 0), _pad1(dst_pa, E_PAD, N_ART)
    sps, dps = _pad1(src_ps, E_PAD, 0), _pad1(dst_ps, E_PAD, N_SONG)
    ssp, dsp = _pad1(src_sp, E_PAD, 0), _pad1(dst_sp, E_PAD, N_PERF)

    c_ap, c_pa, c_ps, c_sp = _hist_kernel()(dap, dpa, dps, dsp)
    rec = lambda c: (1.0 / jnp.maximum(c[0] + c[1], 1.0)).reshape(-1, 1)
    r_ap, r_pa, r_ps, r_sp = rec(c_ap), rec(c_pa), rec(c_ps), rec(c_sp)

    Wls = (Wl0, Wl1, Wl2)
    bls = (bl0, bl1, bl2)
    Wrs = (Wr0, Wr1, Wr2)
    brs = (br0, br1, br2)
    seg_perf = _segsum(128, NPA_PERF)
    seg_sm = _segsum(128, NPA_SM)

    for li in range(3):
        P = len(xp)
        agg_ap = tuple(seg_perf(part, sap, dap) for part in xa)
        agg_sp = tuple(seg_perf(part, ssp, dsp) for part in xs)
        agg_pa = tuple(seg_sm(part, spa, dpa) for part in xp)
        agg_ps = tuple(seg_sm(part, sps, dps) for part in xp)

        Wl, bl, Wr, br = Wls[li], bls[li], Wrs[li], brs[li]
        head = li == 2
        relu = not head
        bsum = lambda e: (bl[e] + br[e]).reshape(1, D_HID)
        head_args = lambda ti: ((ln_g[ti].reshape(1, D_HID),
                                 ln_b[ti].reshape(1, D_HID),
                                 P1[ti], pb1[ti].reshape(1, D_PROJ),
                                 P2[ti], pb2[ti].reshape(1, D_PROJ))
                                if head else ())
        xp = _combine(NPA_PERF, P, 2, relu, head)(
            *xp, *agg_ap, r_ap, Wl[0], Wr[0], bsum(0),
            *agg_sp, r_sp, Wl[3], Wr[3], bsum(3), *head_args(0))
        xa = _combine(NPA_SM, P, 1, relu, head)(
            *xa, *agg_pa, r_pa, Wl[1], Wr[1], bsum(1), *head_args(1))
        xs = _combine(NPA_SM, P, 1, relu, head)(
            *xs, *agg_ps, r_ps, Wl[2], Wr[2], bsum(2), *head_args(2))

    return (xp[:N_PERF], xa[:N_ART], xs[:N_SONG])
